# baseline (speedup 1.0000x reference)
import functools

import jax
import jax.numpy as jnp
from jax import lax
from jax.experimental import pallas as pl
from jax.experimental.pallas import tpu as pltpu

BLANK_ID = 140
NUM_CLASSES = 141
BATCH = 64
SEQ = 4096
SPLIT = 128
REST = NUM_CLASSES - SPLIT


def _tc_body(probs_hbm, idx_ref, keep_ref, bufa, bufb, sema, semb):
    i = pl.program_id(0)
    slot = lax.rem(i, 2)
    nslot = lax.rem(i + 1, 2)

    def start(row, s):
        pltpu.make_async_copy(
            probs_hbm.at[row, :, pl.ds(0, SPLIT)], bufa.at[s], sema.at[s]
        ).start()
        pltpu.make_async_copy(
            probs_hbm.at[row, :, pl.ds(SPLIT, REST)], bufb.at[s], semb.at[s]
        ).start()

    @pl.when(i == 0)
    def _():
        start(0, 0)

    @pl.when(i + 1 < BATCH)
    def _():
        start(i + 1, nslot)

    pltpu.make_async_copy(
        probs_hbm.at[i, :, pl.ds(0, SPLIT)], bufa.at[slot], sema.at[slot]
    ).wait()
    pltpu.make_async_copy(
        probs_hbm.at[i, :, pl.ds(SPLIT, REST)], bufb.at[slot], semb.at[slot]
    ).wait()

    x1 = bufa[slot]
    x2 = bufb[slot]

    m1 = jnp.max(x1, axis=-1, keepdims=True)
    m2 = jnp.max(x2, axis=-1, keepdims=True)
    iota1 = lax.broadcasted_iota(jnp.int32, (SEQ, SPLIT), 1)
    iota2 = lax.broadcasted_iota(jnp.int32, (SEQ, REST), 1)
    a1 = jnp.min(jnp.where(x1 == m1, iota1, NUM_CLASSES), axis=-1, keepdims=True)
    a2 = jnp.min(jnp.where(x2 == m2, iota2, NUM_CLASSES), axis=-1, keepdims=True)
    idx = jnp.where(m1 >= m2, a1, a2 + SPLIT)

    prev = jnp.concatenate(
        [jnp.full((1, 1), -1, jnp.int32), idx[: SEQ - 1, :]], axis=0
    )
    keep = ((idx != prev) & (idx != BLANK_ID)).astype(jnp.int32)

    r = lax.rem(i, 8)
    idx_ref[pl.ds(r, 1), :] = idx.reshape(1, SEQ)
    keep_ref[pl.ds(r, 1), :] = keep.reshape(1, SEQ)


@functools.partial(jax.jit, static_argnums=())
def _tc_call(probs):
    idx, keep = pl.pallas_call(
        _tc_body,
        grid=(BATCH,),
        in_specs=[pl.BlockSpec(memory_space=pl.ANY)],
        out_specs=[
            pl.BlockSpec((8, SEQ), lambda i: (i // 8, 0)),
            pl.BlockSpec((8, SEQ), lambda i: (i // 8, 0)),
        ],
        out_shape=[
            jax.ShapeDtypeStruct((BATCH, SEQ), jnp.int32),
            jax.ShapeDtypeStruct((BATCH, SEQ), jnp.int32),
        ],
        scratch_shapes=[
            pltpu.VMEM((2, SEQ, SPLIT), jnp.float32),
            pltpu.VMEM((2, SEQ, REST), jnp.float32),
            pltpu.SemaphoreType.DMA((2,)),
            pltpu.SemaphoreType.DMA((2,)),
        ],
    )(probs)
    return idx, keep


def kernel(probs):
    idx, keep = _tc_call(probs)
    return idx, keep.astype(bool)


# speedup vs baseline: 1.8485x; 1.8485x over previous
import functools

import jax
import jax.numpy as jnp
from jax import lax
from jax.experimental import pallas as pl
from jax.experimental.pallas import tpu as pltpu

BLANK_ID = 140
NUM_CLASSES = 141
BATCH = 64
SEQ = 4096
SPLIT = 128
REST = NUM_CLASSES - SPLIT


PAD = 128


def _tc_body(probs_hbm, idx_ref, keep_ref, bufa, bufb, shift_buf, sema, semb):
    i = pl.program_id(0)
    slot = lax.rem(i, 2)
    nslot = lax.rem(i + 1, 2)

    def start(row, s):
        pltpu.make_async_copy(
            probs_hbm.at[row, :, pl.ds(0, SPLIT)], bufa.at[s], sema.at[s]
        ).start()
        pltpu.make_async_copy(
            probs_hbm.at[row, :, pl.ds(SPLIT, REST)], bufb.at[s], semb.at[s]
        ).start()

    @pl.when(i == 0)
    def _():
        shift_buf[pl.ds(0, PAD)] = jnp.full((PAD,), -1, jnp.int32)
        start(0, 0)

    @pl.when(i + 1 < BATCH)
    def _():
        start(i + 1, nslot)

    pltpu.make_async_copy(
        probs_hbm.at[i, :, pl.ds(0, SPLIT)], bufa.at[slot], sema.at[slot]
    ).wait()
    pltpu.make_async_copy(
        probs_hbm.at[i, :, pl.ds(SPLIT, REST)], bufb.at[slot], semb.at[slot]
    ).wait()

    x1 = bufa[slot]
    x2 = bufb[slot]

    x1t = x1.T
    x2t = x2.T
    m1 = jnp.max(x1t, axis=0)
    m2 = jnp.max(x2t, axis=0)
    a1 = jnp.argmax(x1t, axis=0)
    a2 = jnp.argmax(x2t, axis=0) + SPLIT
    idx = jnp.where(m1 >= m2, a1, a2)

    shift_buf[pl.ds(PAD, SEQ)] = idx
    prev = shift_buf[pl.ds(PAD - 1, SEQ)]
    keep = ((idx != prev) & (idx != BLANK_ID)).astype(jnp.int32)

    r = lax.rem(i, 8)
    idx_ref[pl.ds(r, 1), :] = idx.reshape(1, SEQ)
    keep_ref[pl.ds(r, 1), :] = keep.reshape(1, SEQ)


@functools.partial(jax.jit, static_argnums=())
def _tc_call(probs):
    idx, keep = pl.pallas_call(
        _tc_body,
        grid=(BATCH,),
        in_specs=[pl.BlockSpec(memory_space=pl.ANY)],
        out_specs=[
            pl.BlockSpec((8, SEQ), lambda i: (i // 8, 0)),
            pl.BlockSpec((8, SEQ), lambda i: (i // 8, 0)),
        ],
        out_shape=[
            jax.ShapeDtypeStruct((BATCH, SEQ), jnp.int32),
            jax.ShapeDtypeStruct((BATCH, SEQ), jnp.int32),
        ],
        scratch_shapes=[
            pltpu.VMEM((2, SEQ, SPLIT), jnp.float32),
            pltpu.VMEM((2, SEQ, REST), jnp.float32),
            pltpu.VMEM((PAD + SEQ,), jnp.int32),
            pltpu.SemaphoreType.DMA((2,)),
            pltpu.SemaphoreType.DMA((2,)),
        ],
    )(probs)
    return idx, keep


def kernel(probs):
    idx, keep = _tc_call(probs)
    return idx, keep.astype(bool)
